# Initial kernel scaffold; baseline (speedup 1.0000x reference)
import jax
import jax.numpy as jnp
from jax.experimental import pallas as pl


def kernel(p3, p4, p5, p6, p7, cls_w, cls_b, cls_gn_g, cls_gn_b, box_w, box_b, box_gn_g, box_gn_b, score_w, score_b, pred_w, pred_b, ctr_w, ctr_b, scales):
    raise NotImplementedError("write your pallas kernel here")


# speedup vs baseline: 2.5566x; 2.5566x over previous
import functools

import numpy as np
import jax
import jax.numpy as jnp
from jax import lax
from jax.experimental import pallas as pl
from jax.experimental.pallas import tpu as pltpu

_C = 256
_NL = 4
_GROUPS = 32
_EPS = 1e-5

_GEOM = []
for _H, _WP, _B in ((64, 72, 80), (32, 40, 48), (16, 24, 32), (8, 16, 24), (4, 8, 16)):
    _NP = (_H + 2) * _WP
    _GEOM.append((_H, _WP, _B, _NP, _NP + 2 * _B))


def _np_mask(H, WP, NP):
    r = np.arange(NP)
    hh, ww = r // WP, r % WP
    m = (hh >= 1) & (hh <= H) & (ww >= 1) & (ww <= H)
    return m.astype(np.float32)[:, None]


_MASKS = [jnp.asarray(_np_mask(H, WP, NP)) for (H, WP, B, NP, NT) in _GEOM]
_AGG = jnp.asarray(
    (np.arange(_C)[:, None] // (_C // _GROUPS) == np.arange(_C)[None, :] // (_C // _GROUPS)
     ).astype(np.float32))


def _taps(WP):
    return [dh * WP + dw for dh in (-1, 0, 1) for dw in (-1, 0, 1)]


def _conv9(x, w_ref, row0, B, NP, WP):
    acc = None
    for t, off in enumerate(_taps(WP)):
        xs = lax.slice(x, (B + off, 0), (B + off + NP, _C))
        wt = w_ref[row0 + t * _C: row0 + (t + 1) * _C, :]
        p = jnp.dot(xs, wt, preferred_element_type=jnp.float32)
        acc = p if acc is None else acc + p
    return acc


def _tower_body(is_box, x_refs, m_refs, agg_ref, w_ref, b_ref, g_ref, bt_ref,
                hw_ref, hb_ref, sc_ref, out_refs):
    aggm = agg_ref[:, :]
    for l in range(5):
        H, WP, B, NP, NT = _GEOM[l]
        mask = m_refs[l][:, :]
        cur = x_refs[l][:, :]
        n = float((_C // _GROUPS) * H * H)
        for i in range(_NL):
            o = _conv9(cur, w_ref, (i * 9) * _C, B, NP, WP)
            o = (o + b_ref[i:i + 1, :]) * mask
            csum = jnp.sum(o, axis=0, keepdims=True)
            csq = jnp.sum(o * o, axis=0, keepdims=True)
            mu = jnp.dot(csum, aggm, preferred_element_type=jnp.float32) / n
            ex2 = jnp.dot(csq, aggm, preferred_element_type=jnp.float32) / n
            s = lax.rsqrt(ex2 - mu * mu + _EPS) * g_ref[i:i + 1, :]
            sh = bt_ref[i:i + 1, :] - mu * s
            o = jnp.maximum(o * s + sh, 0.0) * mask
            z = jnp.zeros((B, _C), jnp.float32)
            cur = jnp.concatenate([z, o, z], axis=0)
        y = _conv9(cur, hw_ref, 0, B, NP, WP) + hb_ref[0:1, :]
        if is_box:
            sval = sc_ref[l:l + 1, :]
            colid = lax.broadcasted_iota(jnp.int32, y.shape, 1)
            y = jnp.where(colid < 4, jnp.maximum(y * sval, 0.0), y)
        out_refs[l][:, :] = y


def _make_body(is_box):
    def body(*refs):
        xs, ms = list(refs[0:5]), list(refs[5:10])
        agg, w, b, g, bt, hw, hb = refs[10:17]
        if is_box:
            sc = refs[17]
            outs = list(refs[18:23])
        else:
            sc = None
            outs = list(refs[17:22])
        _tower_body(is_box, xs, ms, agg, w, b, g, bt, hw, hb, sc, outs)
    return body


def _run_tower(is_box, xps, wm, b, g, bt, hwm, hb, scales2, head_width):
    out_shape = [jax.ShapeDtypeStruct((NP, head_width), jnp.float32)
                 for (_, _, _, NP, _) in _GEOM]
    args = xps + _MASKS + [_AGG, wm, b, g, bt, hwm, hb]
    if is_box:
        args.append(scales2)
    return pl.pallas_call(
        _make_body(is_box),
        out_shape=out_shape,
        compiler_params=pltpu.CompilerParams(
            vmem_limit_bytes=100 * 1024 * 1024),
    )(*args)


def _to_matmul_w(w):
    if w.ndim == 5:
        nl = w.shape[0]
        return w.transpose(0, 3, 4, 2, 1).reshape(nl * 9 * _C, w.shape[1])
    return w.transpose(2, 3, 1, 0).reshape(9 * _C, w.shape[0])


def _pad_level(x, l):
    H, WP, B, NP, NT = _GEOM[l]
    xhwc = x[0].transpose(1, 2, 0)
    buf = jnp.zeros((H + 2, WP, _C), jnp.float32)
    buf = buf.at[1:H + 1, 1:H + 1, :].set(xhwc)
    return jnp.pad(buf.reshape(NP, _C), ((B, B), (0, 0)))


def _extract(y, l, cols):
    H, WP, B, NP, NT = _GEOM[l]
    img = y.reshape(H + 2, WP, -1)[1:H + 1, 1:H + 1, :]
    return img.transpose(2, 0, 1)[None, :cols]


def kernel(p3, p4, p5, p6, p7, cls_w, cls_b, cls_gn_g, cls_gn_b,
           box_w, box_b, box_gn_g, box_gn_b,
           score_w, score_b, pred_w, pred_b, ctr_w, ctr_b, scales):
    feats = [p3, p4, p5, p6, p7]
    xps = [_pad_level(f, l) for l, f in enumerate(feats)]

    clsm = _to_matmul_w(cls_w)
    boxm = _to_matmul_w(box_w)
    scorem = _to_matmul_w(score_w)
    bpm = jnp.concatenate([_to_matmul_w(pred_w), _to_matmul_w(ctr_w)], axis=1)
    scb = score_b[None, :]
    bpb = jnp.concatenate([pred_b, ctr_b])[None, :]
    scales2 = scales[:, None]

    logits_p = _run_tower(False, xps, clsm, cls_b, cls_gn_g, cls_gn_b,
                          scorem, scb, None, 80)
    bc_p = _run_tower(True, xps, boxm, box_b, box_gn_g, box_gn_b,
                      bpm, bpb, scales2, 5)

    logits = [_extract(logits_p[l], l, 80) for l in range(5)]
    bbox = [_extract(bc_p[l][:, 0:4], l, 4) for l in range(5)]
    ctr = [_extract(bc_p[l][:, 4:5], l, 1) for l in range(5)]
    return tuple(logits + bbox + ctr)


# speedup vs baseline: 2.5605x; 1.0015x over previous
import functools

import numpy as np
import jax
import jax.numpy as jnp
from jax import lax
from jax.experimental import pallas as pl
from jax.experimental.pallas import tpu as pltpu

_C = 256
_NL = 4
_GROUPS = 32
_EPS = 1e-5

_GEOM = []
for _H, _WP, _B in ((64, 72, 80), (32, 40, 48), (16, 24, 32), (8, 16, 24), (4, 8, 16)):
    _NP = (_H + 2) * _WP
    _GEOM.append((_H, _WP, _B, _NP, _NP + 2 * _B))


def _np_mask(H, WP, NP):
    r = np.arange(NP)
    hh, ww = r // WP, r % WP
    m = (hh >= 1) & (hh <= H) & (ww >= 1) & (ww <= H)
    return m.astype(np.float32)[:, None]


_MASKS_NP = [_np_mask(H, WP, NP) for (H, WP, B, NP, NT) in _GEOM]
_AGG_NP = (np.arange(_C)[:, None] // (_C // _GROUPS)
           == np.arange(_C)[None, :] // (_C // _GROUPS)).astype(np.float32)


def _shift3(x, B, NP, WP):
    L = NP + 2 * WP
    s = B - WP
    return tuple(lax.slice(x, (s + dw, 0), (s + dw + L, _C))
                 for dw in (-1, 0, 1))


def _conv9(tri, w_ref, row0, NP, WP):
    acc = None
    t = 0
    for dh in (-1, 0, 1):
        r0 = (dh + 1) * WP
        for dw in (-1, 0, 1):
            xs = lax.slice(tri[dw + 1], (r0, 0), (r0 + NP, _C))
            wt = w_ref[row0 + t * _C: row0 + (t + 1) * _C, :]
            p = jnp.dot(xs, wt, preferred_element_type=jnp.float32)
            acc = p if acc is None else acc + p
            t += 1
    return acc


def _tower_body(is_box, x_refs, m_refs, agg_ref, w_ref, b_ref, g_ref, bt_ref,
                hw_ref, hb_ref, sc_ref, out_refs):
    aggm = agg_ref[:, :]
    for l in range(5):
        H, WP, B, NP, NT = _GEOM[l]
        mask = m_refs[l][:, :]
        cur = x_refs[l][:, :]
        n = float((_C // _GROUPS) * H * H)
        for i in range(_NL):
            o = _conv9(_shift3(cur, B, NP, WP), w_ref, (i * 9) * _C, NP, WP)
            o = (o + b_ref[i:i + 1, :]) * mask
            csum = jnp.sum(o, axis=0, keepdims=True)
            csq = jnp.sum(o * o, axis=0, keepdims=True)
            mu = jnp.dot(csum, aggm, preferred_element_type=jnp.float32) / n
            ex2 = jnp.dot(csq, aggm, preferred_element_type=jnp.float32) / n
            s = lax.rsqrt(ex2 - mu * mu + _EPS) * g_ref[i:i + 1, :]
            sh = bt_ref[i:i + 1, :] - mu * s
            o = jnp.maximum(o * s + sh, 0.0) * mask
            z = jnp.zeros((B, _C), jnp.float32)
            cur = jnp.concatenate([z, o, z], axis=0)
        y = _conv9(_shift3(cur, B, NP, WP), hw_ref, 0, NP, WP) + hb_ref[0:1, :]
        if is_box:
            sval = sc_ref[l:l + 1, :]
            colid = lax.broadcasted_iota(jnp.int32, y.shape, 1)
            y = jnp.where(colid < 4, jnp.maximum(y * sval, 0.0), y)
        out_refs[l][:, :] = y


def _make_body(is_box):
    def body(*refs):
        xs, ms = list(refs[0:5]), list(refs[5:10])
        agg, w, b, g, bt, hw, hb = refs[10:17]
        if is_box:
            sc = refs[17]
            outs = list(refs[18:23])
        else:
            sc = None
            outs = list(refs[17:22])
        _tower_body(is_box, xs, ms, agg, w, b, g, bt, hw, hb, sc, outs)
    return body


def _run_tower(is_box, xps, wm, b, g, bt, hwm, hb, scales2, head_width):
    out_shape = [jax.ShapeDtypeStruct((NP, head_width), jnp.float32)
                 for (_, _, _, NP, _) in _GEOM]
    args = xps + [jnp.asarray(m) for m in _MASKS_NP] + [
        jnp.asarray(_AGG_NP), wm, b, g, bt, hwm, hb]
    if is_box:
        args.append(scales2)
    return pl.pallas_call(
        _make_body(is_box),
        out_shape=out_shape,
        compiler_params=pltpu.CompilerParams(
            vmem_limit_bytes=100 * 1024 * 1024),
    )(*args)


def _to_matmul_w(w):
    if w.ndim == 5:
        nl = w.shape[0]
        return w.transpose(0, 3, 4, 2, 1).reshape(nl * 9 * _C, w.shape[1])
    return w.transpose(2, 3, 1, 0).reshape(9 * _C, w.shape[0])


def _pad_level(x, l):
    H, WP, B, NP, NT = _GEOM[l]
    xhwc = x[0].transpose(1, 2, 0)
    buf = jnp.zeros((H + 2, WP, _C), jnp.float32)
    buf = buf.at[1:H + 1, 1:H + 1, :].set(xhwc)
    return jnp.pad(buf.reshape(NP, _C), ((B, B), (0, 0)))


def _extract(y, l, cols):
    H, WP, B, NP, NT = _GEOM[l]
    img = y.reshape(H + 2, WP, -1)[1:H + 1, 1:H + 1, :]
    return img.transpose(2, 0, 1)[None, :cols]


def kernel(p3, p4, p5, p6, p7, cls_w, cls_b, cls_gn_g, cls_gn_b,
           box_w, box_b, box_gn_g, box_gn_b,
           score_w, score_b, pred_w, pred_b, ctr_w, ctr_b, scales):
    feats = [p3, p4, p5, p6, p7]
    xps = [_pad_level(f, l) for l, f in enumerate(feats)]

    clsm = _to_matmul_w(cls_w)
    boxm = _to_matmul_w(box_w)
    scorem = _to_matmul_w(score_w)
    bpm = jnp.concatenate([_to_matmul_w(pred_w), _to_matmul_w(ctr_w)], axis=1)
    scb = score_b[None, :]
    bpb = jnp.concatenate([pred_b, ctr_b])[None, :]
    scales2 = scales[:, None]

    logits_p = _run_tower(False, xps, clsm, cls_b, cls_gn_g, cls_gn_b,
                          scorem, scb, None, 80)
    bc_p = _run_tower(True, xps, boxm, box_b, box_gn_g, box_gn_b,
                      bpm, bpb, scales2, 5)

    logits = [_extract(logits_p[l], l, 80) for l in range(5)]
    bbox = [_extract(bc_p[l][:, 0:4], l, 4) for l in range(5)]
    ctr = [_extract(bc_p[l][:, 4:5], l, 1) for l in range(5)]
    return tuple(logits + bbox + ctr)


# speedup vs baseline: 2.9354x; 1.1464x over previous
import numpy as np
import jax
import jax.numpy as jnp
from jax import lax
from jax.experimental import pallas as pl
from jax.experimental.pallas import tpu as pltpu

_C = 256
_NL = 4
_GROUPS = 32
_EPS = 1e-5

_GEOM = []
for _H, _B in ((64, 72), (32, 40), (16, 24), (8, 16), (4, 8)):
    _NP = (_H + 2) * _H
    _GEOM.append((_H, _B, _NP, _NP + 2 * _B, _NP + 2 * _H))


def _np_shift_masks(H, L):
    i = np.arange(L)
    mm = (i % H != 0).astype(np.float32)[:, None]
    mp = (i % H != H - 1).astype(np.float32)[:, None]
    return mm, mp


_SHIFT_MASKS_NP = [_np_shift_masks(H, L) for (H, B, NP, NT, L) in _GEOM]
_AGG_NP = (np.arange(_C)[:, None] // (_C // _GROUPS)
           == np.arange(_C)[None, :] // (_C // _GROUPS)).astype(np.float32)


def _shift3(x, mm, mp, B, NP, W):
    L = NP + 2 * W
    s = B - W
    xm = lax.slice(x, (s - 1, 0), (s - 1 + L, _C)) * mm
    x0 = lax.slice(x, (s, 0), (s + L, _C))
    xp = lax.slice(x, (s + 1, 0), (s + 1 + L, _C)) * mp
    return (xm, x0, xp)


def _conv9(tri, w_ref, row0, NP, W):
    acc = None
    t = 0
    for dh in (-1, 0, 1):
        r0 = (dh + 1) * W
        for dw in (-1, 0, 1):
            xs = lax.slice(tri[dw + 1], (r0, 0), (r0 + NP, _C))
            wt = w_ref[row0 + t * _C: row0 + (t + 1) * _C, :]
            p = jnp.dot(xs, wt, preferred_element_type=jnp.float32)
            acc = p if acc is None else acc + p
            t += 1
    return acc


def _tower_body(is_box, x_refs, mm_refs, mp_refs, agg_ref, w_ref, b_ref,
                g_ref, bt_ref, hw_ref, hb_ref, sc_ref, out_refs):
    aggm = agg_ref[:, :]
    for l in range(5):
        H, B, NP, NT, L = _GEOM[l]
        HW = H * H
        mm = mm_refs[l][:, :]
        mp = mp_refs[l][:, :]
        z = jnp.zeros((B + H, _C), jnp.float32)
        cur = jnp.concatenate([z, x_refs[l][:, :].T, z], axis=0)
        n = float((_C // _GROUPS) * HW)
        for i in range(_NL):
            o = _conv9(_shift3(cur, mm, mp, B, NP, H), w_ref, (i * 9) * _C,
                       NP, H)
            o = lax.slice(o, (H, 0), (H + HW, _C)) + b_ref[i:i + 1, :]
            csum = jnp.sum(o, axis=0, keepdims=True)
            csq = jnp.sum(o * o, axis=0, keepdims=True)
            mu = jnp.dot(csum, aggm, preferred_element_type=jnp.float32) / n
            ex2 = jnp.dot(csq, aggm, preferred_element_type=jnp.float32) / n
            s = lax.rsqrt(ex2 - mu * mu + _EPS) * g_ref[i:i + 1, :]
            sh = bt_ref[i:i + 1, :] - mu * s
            o = jnp.maximum(o * s + sh, 0.0)
            cur = jnp.concatenate([z, o, z], axis=0)
        y = _conv9(_shift3(cur, mm, mp, B, NP, H), hw_ref, 0, NP, H)
        y = lax.slice(y, (H, 0), (H + HW, y.shape[1])) + hb_ref[0:1, :]
        if is_box:
            sval = sc_ref[l:l + 1, :]
            colid = lax.broadcasted_iota(jnp.int32, y.shape, 1)
            y = jnp.where(colid < 4, jnp.maximum(y * sval, 0.0), y)
        out_refs[l][:, :] = y


def _make_body(is_box):
    def body(*refs):
        xs = list(refs[0:5])
        mms = [refs[5 + 2 * l] for l in range(5)]
        mps = [refs[6 + 2 * l] for l in range(5)]
        agg, w, b, g, bt, hw, hb = refs[15:22]
        if is_box:
            sc = refs[22]
            outs = list(refs[23:28])
        else:
            sc = None
            outs = list(refs[22:27])
        _tower_body(is_box, xs, mms, mps, agg, w, b, g, bt, hw, hb, sc, outs)
    return body


def _run_tower(is_box, xcs, wm, b, g, bt, hwm, hb, scales2, head_width):
    out_shape = [jax.ShapeDtypeStruct((H * H, head_width), jnp.float32)
                 for (H, B, NP, NT, L) in _GEOM]
    args = list(xcs)
    for mm, mp in _SHIFT_MASKS_NP:
        args += [jnp.asarray(mm), jnp.asarray(mp)]
    args += [jnp.asarray(_AGG_NP), wm, b, g, bt, hwm, hb]
    if is_box:
        args.append(scales2)
    return pl.pallas_call(
        _make_body(is_box),
        out_shape=out_shape,
        compiler_params=pltpu.CompilerParams(
            vmem_limit_bytes=100 * 1024 * 1024),
    )(*args)


def _to_matmul_w(w):
    if w.ndim == 5:
        nl = w.shape[0]
        return w.transpose(0, 3, 4, 2, 1).reshape(nl * 9 * _C, w.shape[1])
    return w.transpose(2, 3, 1, 0).reshape(9 * _C, w.shape[0])


def kernel(p3, p4, p5, p6, p7, cls_w, cls_b, cls_gn_g, cls_gn_b,
           box_w, box_b, box_gn_g, box_gn_b,
           score_w, score_b, pred_w, pred_b, ctr_w, ctr_b, scales):
    feats = [p3, p4, p5, p6, p7]
    xcs = [f.reshape(_C, _GEOM[l][0] ** 2) for l, f in enumerate(feats)]

    clsm = _to_matmul_w(cls_w)
    boxm = _to_matmul_w(box_w)
    scorem = _to_matmul_w(score_w)
    bpm = jnp.concatenate([_to_matmul_w(pred_w), _to_matmul_w(ctr_w)], axis=1)
    scb = score_b[None, :]
    bpb = jnp.concatenate([pred_b, ctr_b])[None, :]
    scales2 = scales[:, None]

    logits_i = _run_tower(False, xcs, clsm, cls_b, cls_gn_g, cls_gn_b,
                          scorem, scb, None, 80)
    bc_i = _run_tower(True, xcs, boxm, box_b, box_gn_g, box_gn_b,
                      bpm, bpb, scales2, 5)

    def _img(y, l, cols):
        H = _GEOM[l][0]
        return y.reshape(H, H, -1).transpose(2, 0, 1)[None, :cols]

    logits = [_img(logits_i[l], l, 80) for l in range(5)]
    bbox = [_img(bc_i[l][:, 0:4], l, 4) for l in range(5)]
    ctr = [_img(bc_i[l][:, 4:5], l, 1) for l in range(5)]
    return tuple(logits + bbox + ctr)
